# baseline (speedup 1.0000x reference)
import dataclasses
import math

import jax
import jax.numpy as jnp
from jax import lax
from jax.experimental import pallas as pl
from jax.experimental.pallas import tpu as pltpu
from jax.experimental.pallas import tpu_sc as plsc

D_MODEL = 32
LANES = 16
SCALE = math.sqrt(D_MODEL)
NW = 32
BATCH = 4096
HIST = 200
NBT = BATCH // 128
UNITS = HIST * NBT
UPW = UNITS // NW
IPW = UPW * 128


def _compiler_params():
    cp = pltpu.CompilerParams(use_tc_tiling_on_sc=False)
    if "needs_layout_passes" in pltpu.CompilerParams.__dataclass_fields__:
        cp = dataclasses.replace(cp, needs_layout_passes=False)
    return cp


def _tc_prescale_linearize(lut_t):
    c_blk = 8192
    n_blk = -(-1000000 // c_blk)

    def body(in_ref, out_ref):
        for a in range(4):
            out_ref[:, a * 32:(a + 1) * 32] = (
                in_ref[:, a * 2048:(a + 1) * 2048] * SCALE
            ).T

    return pl.pallas_call(
        body,
        grid=(n_blk,),
        in_specs=[pl.BlockSpec((32, c_blk), lambda i: (0, i))],
        out_specs=pl.BlockSpec((c_blk // 4, 128), lambda i: (i, 0)),
        out_shape=jax.ShapeDtypeStruct((n_blk * c_blk // 4, 128), jnp.float32),
    )(lut_t)


def _sc_gather_scale(xf, lut):
    mesh = plsc.VectorSubcoreMesh(core_axis_name="c", subcore_axis_name="s")

    @pl.kernel(
        out_type=jax.ShapeDtypeStruct((HIST, 4, NBT, 1024), jnp.float32),
        mesh=mesh,
        scratch_types=[
            pltpu.VMEM((IPW,), jnp.int32),
        ]
        + [pltpu.VMEM((128, D_MODEL), jnp.float32)] * 8
        + [pltpu.VMEM((4096,), jnp.float32)] * 4
        + [pltpu.SemaphoreType.DMA] * 12,
        compiler_params=_compiler_params(),
    )
    def kernel_fn(lut_hbm, xf_hbm, out_hbm, slab,
                  r0, r1, r2, r3, r4, r5, r6, r7, s0, s1, s2, s3,
                  g0, g1, g2, g3, g4, g5, g6, g7, o0, o1, o2, o3):
        rows = (r0, r1, r2, r3, r4, r5, r6, r7)
        stg = (s0, s1, s2, s3)
        gsem = (g0, g1, g2, g3, g4, g5, g6, g7)
        osem = (o0, o1, o2, o3)
        wid = lax.axis_index("s") * 2 + lax.axis_index("c")
        u0 = wid * UPW
        iota = lax.iota(jnp.int32, LANES)
        cbase = ((iota >> 3) << 10) + ((iota & 7) << 7)

        def gather_copy(t, j):
            return pltpu.make_async_copy(
                lut_hbm.at[slab.at[pl.ds(t * 128, 128)]], rows[j], gsem[j]
            )

        def out_copy(u, dt, p):
            return pltpu.make_async_copy(
                stg[p].at[pl.ds(dt * 1024, 1024)],
                out_hbm.at[u // NBT, dt, u % NBT],
                osem[p],
            )

        pltpu.sync_copy(xf_hbm.at[pl.ds(u0 * 128, IPW)], slab)

        @pl.loop(0, IPW, step=64)
        def _(s):
            for q in range(4):
                off = s + LANES * q
                v = slab.at[pl.ds(off, LANES)][...]
                j2 = (v & -8192) + ((v & 2047) << 2) + ((v >> 11) & 3)
                slab.at[pl.ds(off, LANES)][...] = j2

        for j in range(8):
            gather_copy(j, j).start()

        @pl.loop(0, UPW, step=8)
        def _(t):
            for j in range(8):
                tt = t + j
                u = u0 + tt
                p = j % 4

                gather_copy(tt, j).wait()

                @pl.when(tt >= 4)
                def _():
                    for dt in range(4):
                        out_copy(u - 4, dt, p).wait()

                @pl.loop(0, 128, step=4)
                def _(bl):
                    for q in range(4):
                        b = bl + q
                        dst = cbase + b
                        v0 = rows[j].at[b, pl.ds(0, LANES)][...]
                        plsc.store_scatter(stg[p], [dst], v0)
                        v1 = rows[j].at[b, pl.ds(LANES, LANES)][...]
                        plsc.store_scatter(stg[p], [dst + 2048], v1)

                @pl.when(tt + 8 < UPW)
                def _():
                    gather_copy(tt + 8, j).start()

                for dt in range(4):
                    out_copy(u, dt, p).start()

        for q in range(4):
            for dt in range(4):
                out_copy(u0 + UPW - 4 + q, dt, q).wait()

    return kernel_fn(lut, xf)


@jax.jit
def kernel(x, lut):
    xf = x.astype(jnp.int32).T.reshape(BATCH * HIST)
    lut_lin = _tc_prescale_linearize(lut.T).reshape(-1, D_MODEL)
    out_lin = _sc_gather_scale(xf, lut_lin)
    return (
        out_lin.reshape(HIST, 4, NBT, 8, 128)
        .transpose(2, 4, 0, 1, 3)
        .reshape(BATCH, HIST, D_MODEL)
    )


# speedup vs baseline: 1.0818x; 1.0818x over previous
import dataclasses
import math

import jax
import jax.numpy as jnp
from jax import lax
from jax.experimental import pallas as pl
from jax.experimental.pallas import tpu as pltpu
from jax.experimental.pallas import tpu_sc as plsc

D_MODEL = 32
LANES = 16
SCALE = math.sqrt(D_MODEL)
NW = 32
BATCH = 4096
HIST = 200
NBT = BATCH // 128
UNITS = HIST * NBT
UPW = UNITS // NW
IPW = UPW * 128


def _compiler_params():
    cp = pltpu.CompilerParams(use_tc_tiling_on_sc=False)
    if "needs_layout_passes" in pltpu.CompilerParams.__dataclass_fields__:
        cp = dataclasses.replace(cp, needs_layout_passes=False)
    return cp


def _tc_prescale_linearize(lut_t):
    c_blk = 8192
    n_blk = -(-1000000 // c_blk)

    def body(in_ref, out_ref):
        for a in range(4):
            out_ref[:, a * 32:(a + 1) * 32] = (
                in_ref[:, a * 2048:(a + 1) * 2048] * SCALE
            ).T

    return pl.pallas_call(
        body,
        grid=(n_blk,),
        in_specs=[pl.BlockSpec((32, c_blk), lambda i: (0, i))],
        out_specs=pl.BlockSpec((c_blk // 4, 128), lambda i: (i, 0)),
        out_shape=jax.ShapeDtypeStruct((n_blk * c_blk // 4, 128), jnp.float32),
    )(lut_t)


def _sc_gather_scale(xf, lut):
    mesh = plsc.VectorSubcoreMesh(core_axis_name="c", subcore_axis_name="s")

    @pl.kernel(
        out_type=jax.ShapeDtypeStruct((HIST, 4, NBT, 1024), jnp.float32),
        mesh=mesh,
        scratch_types=[
            pltpu.VMEM((IPW,), jnp.int32),
        ]
        + [pltpu.VMEM((128, D_MODEL), jnp.float32)] * 8
        + [pltpu.VMEM((4096,), jnp.float32)] * 4
        + [pltpu.SemaphoreType.DMA] * 12,
        compiler_params=_compiler_params(),
    )
    def kernel_fn(lut_hbm, xf_hbm, out_hbm, slab,
                  r0, r1, r2, r3, r4, r5, r6, r7, s0, s1, s2, s3,
                  g0, g1, g2, g3, g4, g5, g6, g7, o0, o1, o2, o3):
        rows = (r0, r1, r2, r3, r4, r5, r6, r7)
        stg = (s0, s1, s2, s3)
        gsem = (g0, g1, g2, g3, g4, g5, g6, g7)
        osem = (o0, o1, o2, o3)
        wid = lax.axis_index("s") * 2 + lax.axis_index("c")
        u0 = wid * UPW
        iota = lax.iota(jnp.int32, LANES)
        cbase = ((iota >> 3) << 10) + ((iota & 7) << 7)

        def gather_copy(t, j):
            return pltpu.make_async_copy(
                lut_hbm.at[slab.at[pl.ds(t * 128, 128)]], rows[j], gsem[j]
            )

        def out_copy(u, dt, p):
            return pltpu.make_async_copy(
                stg[p].at[pl.ds(dt * 1024, 1024)],
                out_hbm.at[u // NBT, dt, u % NBT],
                osem[p],
            )

        pltpu.sync_copy(xf_hbm.at[pl.ds(u0 * 128, IPW)], slab)

        @pl.loop(0, IPW, step=64)
        def _(s):
            for q in range(4):
                off = s + LANES * q
                v = slab.at[pl.ds(off, LANES)][...]
                j2 = (v & -8192) + ((v & 2047) << 2) + ((v >> 11) & 3)
                slab.at[pl.ds(off, LANES)][...] = j2

        for j in range(8):
            gather_copy(j, j).start()

        @pl.loop(0, UPW, step=8)
        def _(t):
            for j in range(8):
                tt = t + j
                u = u0 + tt
                p = j % 4

                gather_copy(tt, j).wait()

                @pl.when(tt >= 4)
                def _():
                    for dt in range(4):
                        out_copy(u - 4, dt, p).wait()

                @pl.loop(0, 128, step=8)
                def _(bl):
                    vs = []
                    for q in range(8):
                        b = bl + q
                        vs.append(
                            (
                                rows[j].at[b, pl.ds(0, LANES)][...],
                                rows[j].at[b, pl.ds(LANES, LANES)][...],
                            )
                        )
                    dstb = cbase + bl
                    for q in range(8):
                        v0, v1 = vs[q]
                        plsc.store_scatter(stg[p], [dstb + q], v0)
                        plsc.store_scatter(stg[p], [dstb + (q + 2048)], v1)

                @pl.when(tt + 8 < UPW)
                def _():
                    gather_copy(tt + 8, j).start()

                for dt in range(4):
                    out_copy(u, dt, p).start()

        for q in range(4):
            for dt in range(4):
                out_copy(u0 + UPW - 4 + q, dt, q).wait()

    return kernel_fn(lut, xf)


@jax.jit
def kernel(x, lut):
    xf = x.astype(jnp.int32).T.reshape(BATCH * HIST)
    lut_lin = _tc_prescale_linearize(lut.T).reshape(-1, D_MODEL)
    out_lin = _sc_gather_scale(xf, lut_lin)
    return (
        out_lin.reshape(HIST, 4, NBT, 8, 128)
        .transpose(2, 4, 0, 1, 3)
        .reshape(BATCH, HIST, D_MODEL)
    )


# speedup vs baseline: 2.2386x; 2.0694x over previous
import dataclasses
import math

import jax
import jax.numpy as jnp
from jax import lax
from jax.experimental import pallas as pl
from jax.experimental.pallas import tpu as pltpu
from jax.experimental.pallas import tpu_sc as plsc

D_MODEL = 32
LANES = 16
SCALE = math.sqrt(D_MODEL)
NW = 32
BATCH = 4096
HIST = 200
NBT = BATCH // 128
UNITS = HIST * NBT
UPW = UNITS // NW
IPW = UPW * 128


def _compiler_params():
    cp = pltpu.CompilerParams(use_tc_tiling_on_sc=False)
    if "needs_layout_passes" in pltpu.CompilerParams.__dataclass_fields__:
        cp = dataclasses.replace(cp, needs_layout_passes=False)
    return cp


def _tc_prescale_linearize(lut_t):
    c_blk = 8192
    n_blk = -(-1000000 // c_blk)

    def body(in_ref, out_ref):
        for a in range(4):
            out_ref[:, a * 32:(a + 1) * 32] = (
                in_ref[:, a * 2048:(a + 1) * 2048] * SCALE
            ).T

    return pl.pallas_call(
        body,
        grid=(n_blk,),
        in_specs=[pl.BlockSpec((32, c_blk), lambda i: (0, i))],
        out_specs=pl.BlockSpec((c_blk // 4, 128), lambda i: (i, 0)),
        out_shape=jax.ShapeDtypeStruct((n_blk * c_blk // 4, 128), jnp.float32),
    )(lut_t)


def _sc_gather_scale(xf, lut):
    mesh = plsc.VectorSubcoreMesh(core_axis_name="c", subcore_axis_name="s")

    @pl.kernel(
        out_type=jax.ShapeDtypeStruct((HIST, 4, NBT, 8, 128), jnp.float32),
        mesh=mesh,
        scratch_types=[
            pltpu.VMEM((IPW,), jnp.int32),
        ]
        + [pltpu.VMEM((128, D_MODEL), jnp.float32)] * 8
        + [pltpu.VMEM((4, 8, 129), jnp.float32)] * 4
        + [pltpu.SemaphoreType.DMA] * 12,
        compiler_params=_compiler_params(),
    )
    def kernel_fn(lut_hbm, xf_hbm, out_hbm, slab,
                  r0, r1, r2, r3, r4, r5, r6, r7, s0, s1, s2, s3,
                  g0, g1, g2, g3, g4, g5, g6, g7, o0, o1, o2, o3):
        rows = (r0, r1, r2, r3, r4, r5, r6, r7)
        stg = (s0, s1, s2, s3)
        gsem = (g0, g1, g2, g3, g4, g5, g6, g7)
        osem = (o0, o1, o2, o3)
        wid = lax.axis_index("s") * 2 + lax.axis_index("c")
        u0 = wid * UPW
        iota = lax.iota(jnp.int32, LANES)
        i0lo = iota >> 3
        i1v = iota & 7

        def gather_copy(t, j):
            return pltpu.make_async_copy(
                lut_hbm.at[slab.at[pl.ds(t * 128, 128)]], rows[j], gsem[j]
            )

        def out_copy(u, dt, p):
            return pltpu.make_async_copy(
                stg[p].at[dt, pl.ds(0, 8), pl.ds(0, 128)],
                out_hbm.at[u // NBT, dt, u % NBT],
                osem[p],
            )

        pltpu.sync_copy(xf_hbm.at[pl.ds(u0 * 128, IPW)], slab)

        @pl.loop(0, IPW, step=64)
        def _(s):
            for q in range(4):
                off = s + LANES * q
                v = slab.at[pl.ds(off, LANES)][...]
                j2 = (v & -8192) + ((v & 2047) << 2) + ((v >> 11) & 3)
                slab.at[pl.ds(off, LANES)][...] = j2

        for j in range(8):
            gather_copy(j, j).start()

        @pl.loop(0, UPW, step=8)
        def _(t):
            for j in range(8):
                tt = t + j
                u = u0 + tt
                p = j % 4

                gather_copy(tt, j).wait()

                @pl.when(tt >= 4)
                def _():
                    for dt in range(4):
                        out_copy(u - 4, dt, p).wait()

                @pl.loop(0, 128, step=8)
                def _(bl):
                    vs = []
                    for q in range(8):
                        b = bl + q
                        vs.append(
                            (
                                rows[j].at[b, pl.ds(0, LANES)][...],
                                rows[j].at[b, pl.ds(LANES, LANES)][...],
                            )
                        )
                    blv = jnp.full((LANES,), 0, jnp.int32) + bl
                    for q in range(8):
                        v0, v1 = vs[q]
                        plsc.store_scatter(
                            stg[p], [i0lo, i1v, blv + q], v0
                        )
                        plsc.store_scatter(
                            stg[p], [i0lo + 2, i1v, blv + q], v1
                        )

                @pl.when(tt + 8 < UPW)
                def _():
                    gather_copy(tt + 8, j).start()

                for dt in range(4):
                    out_copy(u, dt, p).start()

        for q in range(4):
            for dt in range(4):
                out_copy(u0 + UPW - 4 + q, dt, q).wait()

    return kernel_fn(lut, xf)


@jax.jit
def kernel(x, lut):
    xf = x.astype(jnp.int32).T.reshape(BATCH * HIST)
    lut_lin = _tc_prescale_linearize(lut.T).reshape(-1, D_MODEL)
    out_lin = _sc_gather_scale(xf, lut_lin)
    return out_lin.transpose(2, 4, 0, 1, 3).reshape(BATCH, HIST, D_MODEL)
